# Initial kernel scaffold; baseline (speedup 1.0000x reference)
import functools

import jax
import jax.numpy as jnp
from jax import lax
from jax.experimental import pallas as pl
from jax.experimental.pallas import tpu as pltpu
from jax.experimental.pallas import tpu_sc as plsc

D = 64
V = 100000
PAD = 0
VP = 100352
ZROW = V
SCALE = 8.0

NC, NS, L = 2, 16, 16
NW = NC * NS

K = 128


def _scale_body(t_ref, o_ref):
    o_ref[...] = t_ref[...] * SCALE


def _make_scaled_table(table_padded):
    blk = 1024
    return pl.pallas_call(
        _scale_body,
        out_shape=jax.ShapeDtypeStruct((VP, D), jnp.float32),
        grid=(VP // blk,),
        in_specs=[pl.BlockSpec((blk, D), lambda i: (i, 0))],
        out_specs=pl.BlockSpec((blk, D), lambda i: (i, 0)),
    )(table_padded)


def _gather_body(table_hbm, idx_hbm, out_hbm, idx_v, rows_v, sem):
    n_idx = idx_hbm.shape[0]
    per_w = n_idx // NW
    wid = lax.axis_index("s") * NC + lax.axis_index("c")
    base = wid * per_w

    def step(i, carry):
        off = base + i * K
        pltpu.sync_copy(idx_hbm.at[pl.ds(off, K)], idx_v)
        for j in range(K // L):
            v = idx_v[pl.ds(j * L, L)]
            idx_v[pl.ds(j * L, L)] = jnp.where(v == PAD, ZROW, v)
        pltpu.async_copy(table_hbm.at[idx_v], rows_v, sem).wait()
        pltpu.sync_copy(rows_v, out_hbm.at[pl.ds(off, K)])
        return carry

    lax.fori_loop(0, per_w // K, step, 0)


def kernel(inputs, table):
    b, h = inputs.shape
    n = b * h
    idx = inputs.reshape(n).astype(jnp.int32)
    table_padded = jnp.pad(table, ((0, VP - V), (0, 0)))
    scaled = _make_scaled_table(table_padded)

    mesh = plsc.VectorSubcoreMesh(
        core_axis_name="c", subcore_axis_name="s", num_cores=NC, num_subcores=NS
    )
    gather = functools.partial(
        pl.kernel,
        out_type=jax.ShapeDtypeStruct((n, D), jnp.float32),
        mesh=mesh,
        scratch_types=[
            pltpu.VMEM((K,), jnp.int32),
            pltpu.VMEM((K, D), jnp.float32),
            pltpu.SemaphoreType.DMA,
        ],
    )(_gather_body)

    out = gather(scaled, idx)
    return out.reshape(b, h, D)


# speedup vs baseline: 3.6525x; 3.6525x over previous
import functools

import jax
import jax.numpy as jnp
from jax import lax
from jax.experimental import pallas as pl
from jax.experimental.pallas import tpu as pltpu
from jax.experimental.pallas import tpu_sc as plsc

D = 64
V = 100000
PAD = 0
VP = 100352
ZROW = V
SCALE = 8.0

NC, NS, L = 2, 16, 16
NW = NC * NS

K = 128


def _scale_body(t_ref, o_ref):
    o_ref[...] = t_ref[...] * SCALE


def _make_scaled_table(table_padded):
    blk = 1024
    return pl.pallas_call(
        _scale_body,
        out_shape=jax.ShapeDtypeStruct((VP, D), jnp.float32),
        grid=(VP // blk,),
        in_specs=[pl.BlockSpec((blk, D), lambda i: (i, 0))],
        out_specs=pl.BlockSpec((blk, D), lambda i: (i, 0)),
    )(table_padded)


def _gather_body(table_hbm, idx_hbm, out_hbm, idx_v, rows_v, sem):
    n_idx = idx_hbm.shape[0]
    per_w = n_idx // NW
    wid = lax.axis_index("s") * NC + lax.axis_index("c")
    base = wid * per_w

    def step(i, carry):
        off = base + i * K
        pltpu.sync_copy(idx_hbm.at[pl.ds(off, K)], idx_v)
        for j in range(K // L):
            v = idx_v[pl.ds(j * L, L)]
            idx_v[pl.ds(j * L, L)] = jnp.where(v == PAD, ZROW, v)
        pltpu.async_copy(table_hbm.at[idx_v], rows_v, sem).wait()
        pltpu.sync_copy(rows_v, out_hbm.at[pl.ds(off, K)])
        return carry

    lax.fori_loop(0, per_w // K, step, 0)


def kernel(inputs, table):
    b, h = inputs.shape
    n = b * h
    idx = inputs.reshape(n).astype(jnp.int32)
    table_padded = jnp.pad(table, ((0, VP - V), (0, 0)))
    scaled = _make_scaled_table(table_padded)

    mesh = plsc.VectorSubcoreMesh(
        core_axis_name="c", subcore_axis_name="s", num_cores=NC, num_subcores=NS
    )
    gather = functools.partial(
        pl.kernel,
        out_type=jax.ShapeDtypeStruct((n, D), jnp.float32),
        mesh=mesh,
        scratch_types=[
            pltpu.VMEM((K,), jnp.int32),
            pltpu.VMEM((K, D), jnp.float32),
            pltpu.SemaphoreType.DMA,
        ],
        compiler_params=pltpu.CompilerParams(use_tc_tiling_on_sc=False),
    )(_gather_body)

    out = gather(scaled, idx)
    return out.reshape(b, h, D)


# speedup vs baseline: 4.9210x; 1.3473x over previous
import functools

import jax
import jax.numpy as jnp
from jax import lax
from jax.experimental import pallas as pl
from jax.experimental.pallas import tpu as pltpu
from jax.experimental.pallas import tpu_sc as plsc

D = 64
V = 100000
SCALE = 8.0

NC, NS = 2, 16
NW = NC * NS

CH = 128
NCHUNK = 8
BI = NCHUNK * CH
NSLOT = 4

TBLK = 1000


def _scale_body(t_ref, o_ref):
    i = pl.program_id(0)
    row = lax.broadcasted_iota(jnp.int32, (TBLK, D), 0) + i * TBLK
    o_ref[...] = t_ref[...] * jnp.where(row == 0, 0.0, SCALE)


def _make_scaled_table(table):
    return pl.pallas_call(
        _scale_body,
        out_shape=jax.ShapeDtypeStruct((V, D), jnp.float32),
        grid=(V // TBLK,),
        in_specs=[pl.BlockSpec((TBLK, D), lambda i: (i, 0))],
        out_specs=pl.BlockSpec((TBLK, D), lambda i: (i, 0)),
    )(table)


def _gather_body(table_hbm, idx_hbm, out_hbm,
                 idx0, idx1, r0, r1, r2, r3,
                 is0, is1, gs0, gs1, gs2, gs3, os0, os1, os2, os3):
    idxb = (idx0, idx1)
    rows = (r0, r1, r2, r3)
    isem = (is0, is1)
    gsem = (gs0, gs1, gs2, gs3)
    osem = (os0, os1, os2, os3)

    rows_total = idx_hbm.shape[0]
    per_w = rows_total // NW
    nb = per_w // NCHUNK
    wid = lax.axis_index("s") * NC + lax.axis_index("c")
    base_row = wid * per_w

    def run_block(blk, p):
        row0 = base_row + blk * NCHUNK
        pltpu.make_async_copy(
            idx_hbm.at[pl.ds(row0, NCHUNK)], idxb[p], isem[p]).wait()

        @pl.when(blk + 1 < nb)
        def _():
            pltpu.async_copy(
                idx_hbm.at[pl.ds(row0 + NCHUNK, NCHUNK)],
                idxb[1 - p], isem[1 - p])

        gd = [None] * NCHUNK
        od = [None] * NCHUNK
        for c in range(NCHUNK):
            s = c % NSLOT
            if c < NSLOT:
                @pl.when(blk > 0)
                def _(s=s):
                    pltpu.make_async_copy(
                        rows[s], out_hbm.at[pl.ds(row0 * CH, CH)],
                        osem[s]).wait()
            else:
                od[c - NSLOT].wait()
            gd[c] = pltpu.async_copy(
                table_hbm.at[idxb[p].at[c]], rows[s], gsem[s])
            if c >= 1:
                gd[c - 1].wait()
                od[c - 1] = pltpu.async_copy(
                    rows[(c - 1) % NSLOT],
                    out_hbm.at[pl.ds((row0 + c - 1) * CH, CH)],
                    osem[(c - 1) % NSLOT])
        gd[NCHUNK - 1].wait()
        od[NCHUNK - 1] = pltpu.async_copy(
            rows[(NCHUNK - 1) % NSLOT],
            out_hbm.at[pl.ds((row0 + NCHUNK - 1) * CH, CH)],
            osem[(NCHUNK - 1) % NSLOT])

    pltpu.async_copy(idx_hbm.at[pl.ds(base_row, NCHUNK)], idx0, is0)

    def mbody(m, carry):
        run_block(2 * m, 0)
        run_block(2 * m + 1, 1)
        return carry

    lax.fori_loop(0, nb // 2, mbody, 0)

    for s in range(NSLOT):
        pltpu.make_async_copy(
            rows[s], out_hbm.at[pl.ds(base_row * CH, CH)], osem[s]).wait()


def kernel(inputs, table):
    b, h = inputs.shape
    n = b * h
    idx = inputs.reshape(n // CH, CH).astype(jnp.int32)
    scaled = _make_scaled_table(table)

    mesh = plsc.VectorSubcoreMesh(
        core_axis_name="c", subcore_axis_name="s", num_cores=NC, num_subcores=NS
    )
    gather = functools.partial(
        pl.kernel,
        out_type=jax.ShapeDtypeStruct((n, D), jnp.float32),
        mesh=mesh,
        scratch_types=[
            pltpu.VMEM((NCHUNK, CH), jnp.int32),
            pltpu.VMEM((NCHUNK, CH), jnp.int32),
            pltpu.VMEM((CH, D), jnp.float32),
            pltpu.VMEM((CH, D), jnp.float32),
            pltpu.VMEM((CH, D), jnp.float32),
            pltpu.VMEM((CH, D), jnp.float32),
            pltpu.SemaphoreType.DMA,
            pltpu.SemaphoreType.DMA,
            pltpu.SemaphoreType.DMA,
            pltpu.SemaphoreType.DMA,
            pltpu.SemaphoreType.DMA,
            pltpu.SemaphoreType.DMA,
            pltpu.SemaphoreType.DMA,
            pltpu.SemaphoreType.DMA,
            pltpu.SemaphoreType.DMA,
            pltpu.SemaphoreType.DMA,
        ],
        compiler_params=pltpu.CompilerParams(use_tc_tiling_on_sc=False),
    )(_gather_body)

    out = gather(scaled, idx)
    return out.reshape(b, h, D)


# speedup vs baseline: 5.9775x; 1.2147x over previous
import functools

import jax
import jax.numpy as jnp
from jax import lax
from jax.experimental import pallas as pl
from jax.experimental.pallas import tpu as pltpu
from jax.experimental.pallas import tpu_sc as plsc

D = 64
V = 100000
SCALE = 8.0

NC, NS = 2, 16
NW = NC * NS

B = 16384
H = 200
NBT = B // 128
BT_PER_W = NBT // NW
NRB = 4


def _gather_body(table_hbm, idx_hbm, out_hbm,
                 islab, r0, r1, r2, r3, t0, t1,
                 gs0, gs1, gs2, gs3, ws0, ws1):
    rows = (r0, r1, r2, r3)
    trans = (t0, t1)
    gsem = (gs0, gs1, gs2, gs3)
    wsem = (ws0, ws1)
    wid = lax.axis_index("s") * NC + lax.axis_index("c")

    iota = lax.iota(jnp.int32, 16)
    ri = [iota + 16 * k for k in range(8)]
    rk64 = [(iota + 16 * k) * D for k in range(8)]

    def transpose(p4, p2, f):
        def tbody(j, carry):
            cbj = (iota + j) & 15
            for dd in range(D // 16):
                cc = cbj + 16 * dd
                for k in range(8):
                    v = plsc.load_gather(rows[p4], [ri[k], cc])
                    plsc.store_scatter(trans[p2], [cc, ri[k]], v * f[k])
            return carry
        lax.fori_loop(0, 16, tbody, 0)

    def step(l, p4, bt):
        p2 = p4 % 2
        @pl.when(l >= 2)
        def _():
            pltpu.make_async_copy(
                table_hbm.at[pl.ds(0, 128)], rows[p4], wsem[p2]).wait()
        @pl.when(l + 3 < H)
        def _():
            pltpu.async_copy(
                table_hbm.at[islab.at[l + 3]],
                rows[(p4 + 3) % NRB], gsem[(p4 + 3) % NRB])
        f = [jnp.where(islab[l, pl.ds(16 * k, 16)] == 0, 0.0, SCALE)
             for k in range(8)]
        pltpu.make_async_copy(
            table_hbm.at[islab.at[l]], rows[p4], gsem[p4]).wait()
        transpose(p4, p2, f)
        for dt in range(8):
            pltpu.async_copy(
                trans[p2].at[pl.ds(8 * dt, 8)], out_hbm.at[l, dt, bt],
                wsem[p2])

    def bt_body(j, carry):
        bt = wid * BT_PER_W + j
        pltpu.sync_copy(idx_hbm.at[bt], islab)
        for p in range(3):
            pltpu.async_copy(table_hbm.at[islab.at[p]], rows[p], gsem[p])

        def mbody(m, c2):
            for p4 in range(NRB):
                step(NRB * m + p4, p4, bt)
            return c2

        lax.fori_loop(0, H // NRB, mbody, 0)
        for p in range(2):
            pltpu.make_async_copy(
                table_hbm.at[pl.ds(0, 128)], rows[p], wsem[p]).wait()
        return carry

    lax.fori_loop(0, BT_PER_W, bt_body, 0)


def kernel(inputs, table):
    idx_t = (inputs.astype(jnp.int32).T
             .reshape(H, NBT, 128).transpose(1, 0, 2))

    mesh = plsc.VectorSubcoreMesh(
        core_axis_name="c", subcore_axis_name="s", num_cores=NC, num_subcores=NS
    )
    gather = functools.partial(
        pl.kernel,
        out_type=jax.ShapeDtypeStruct((H, 8, NBT, 8, 128), jnp.float32),
        mesh=mesh,
        scratch_types=[
            pltpu.VMEM((H, 128), jnp.int32),
            pltpu.VMEM((128, D), jnp.float32),
            pltpu.VMEM((128, D), jnp.float32),
            pltpu.VMEM((128, D), jnp.float32),
            pltpu.VMEM((128, D), jnp.float32),
            pltpu.VMEM((D, 128), jnp.float32),
            pltpu.VMEM((D, 128), jnp.float32),
            pltpu.SemaphoreType.DMA,
            pltpu.SemaphoreType.DMA,
            pltpu.SemaphoreType.DMA,
            pltpu.SemaphoreType.DMA,
            pltpu.SemaphoreType.DMA,
            pltpu.SemaphoreType.DMA,
        ],
        compiler_params=pltpu.CompilerParams(
            use_tc_tiling_on_sc=False, needs_layout_passes=False,
            disable_bounds_checks=True),
    )(_gather_body)

    y = gather(table, idx_t)
    return y.transpose(2, 4, 0, 1, 3).reshape(B, H, D)


# speedup vs baseline: 7.4444x; 1.2454x over previous
import functools

import jax
import jax.numpy as jnp
from jax import lax
from jax.experimental import pallas as pl
from jax.experimental.pallas import tpu as pltpu
from jax.experimental.pallas import tpu_sc as plsc

D = 64
V = 100000
SCALE = 8.0

NC, NS = 2, 16
NW = NC * NS

B = 16384
H = 200
NBT = B // 128
BT_PER_W = NBT // NW

TBLK = 1000


def _scale_body(t_ref, o_ref):
    i = pl.program_id(0)
    row = lax.broadcasted_iota(jnp.int32, (TBLK, D), 0) + i * TBLK
    o_ref[...] = t_ref[...] * jnp.where(row == 0, 0.0, SCALE)


def _make_scaled_table(table):
    return pl.pallas_call(
        _scale_body,
        out_shape=jax.ShapeDtypeStruct((V, D), jnp.float32),
        grid=(V // TBLK,),
        in_specs=[pl.BlockSpec((TBLK, D), lambda i: (i, 0))],
        out_specs=pl.BlockSpec((TBLK, D), lambda i: (i, 0)),
    )(table)


NRB = 4


def _gather_body(table_hbm, idx_hbm, out_hbm,
                 islab, r0, r1, r2, r3, t0, t1,
                 gs0, gs1, gs2, gs3, ws0, ws1):
    rows = (r0, r1, r2, r3)
    trans = (t0, t1)
    gsem = (gs0, gs1, gs2, gs3)
    wsem = (ws0, ws1)
    wid = lax.axis_index("s") * NC + lax.axis_index("c")

    iota = lax.iota(jnp.int32, 16)
    ri = [iota + 16 * k for k in range(8)]
    zeros = jnp.zeros((16,), jnp.int32)

    def transpose(p4, p2):
        def tbody(j, carry):
            cbj = (iota + j) & 15
            for dd in range(D // 16):
                ci = cbj + 16 * dd
                dtv = ci >> 3
                div = ci & 7
                for k in range(8):
                    v = plsc.load_gather(rows[p4], [ri[k], ci])
                    plsc.store_scatter(trans[p2], [dtv, div, ri[k]], v)
            return carry
        lax.fori_loop(0, 16, tbody, 0)

    def step(l, p4, bt):
        p2 = p4 % 2
        @pl.when(l >= 2)
        def _():
            pltpu.make_async_copy(
                table_hbm.at[pl.ds(0, 128)], rows[p4], wsem[p2]).wait()
        @pl.when(l + 3 < H)
        def _():
            pltpu.async_copy(
                table_hbm.at[islab.at[l + 3]],
                rows[(p4 + 3) % NRB], gsem[(p4 + 3) % NRB])
        pltpu.make_async_copy(
            table_hbm.at[islab.at[l]], rows[p4], gsem[p4]).wait()
        transpose(p4, p2)
        pltpu.async_copy(trans[p2], out_hbm.at[l, :, bt], wsem[p2])

    def bt_body(j, carry):
        bt = wid * BT_PER_W + j
        pltpu.sync_copy(idx_hbm.at[bt], islab)
        for p in range(3):
            pltpu.async_copy(table_hbm.at[islab.at[p]], rows[p], gsem[p])

        def mbody(m, c2):
            for p4 in range(NRB):
                step(NRB * m + p4, p4, bt)
            return c2

        lax.fori_loop(0, H // NRB, mbody, 0)
        for p in range(2):
            pltpu.make_async_copy(
                table_hbm.at[pl.ds(0, 128)], rows[p], wsem[p]).wait()
        return carry

    lax.fori_loop(0, BT_PER_W, bt_body, 0)


def kernel(inputs, table):
    scaled = _make_scaled_table(table)
    idx_t = (inputs.astype(jnp.int32).T
             .reshape(H, NBT, 128).transpose(1, 0, 2))

    mesh = plsc.VectorSubcoreMesh(
        core_axis_name="c", subcore_axis_name="s", num_cores=NC, num_subcores=NS
    )
    gather = functools.partial(
        pl.kernel,
        out_type=jax.ShapeDtypeStruct((H, 8, NBT, 8, 128), jnp.float32),
        mesh=mesh,
        scratch_types=[
            pltpu.VMEM((H, 128), jnp.int32),
            pltpu.VMEM((128, D), jnp.float32),
            pltpu.VMEM((128, D), jnp.float32),
            pltpu.VMEM((128, D), jnp.float32),
            pltpu.VMEM((128, D), jnp.float32),
            pltpu.VMEM((8, 8, 128), jnp.float32),
            pltpu.VMEM((8, 8, 128), jnp.float32),
            pltpu.SemaphoreType.DMA,
            pltpu.SemaphoreType.DMA,
            pltpu.SemaphoreType.DMA,
            pltpu.SemaphoreType.DMA,
            pltpu.SemaphoreType.DMA,
            pltpu.SemaphoreType.DMA,
        ],
        compiler_params=pltpu.CompilerParams(
            use_tc_tiling_on_sc=False, needs_layout_passes=False,
            disable_bounds_checks=True),
    )(_gather_body)

    y = gather(scaled, idx_t)
    return y.transpose(2, 4, 0, 1, 3).reshape(B, H, D)
